# baseline (speedup 1.0000x reference)
import functools

import jax
import jax.numpy as jnp
from jax import lax
from jax.experimental import pallas as pl
from jax.experimental.pallas import tpu as pltpu
from jax.experimental.pallas import tpu_sc as plsc

NC = 2
NS = 16
NW = NC * NS
K_EDGE = 128


@functools.partial(jax.jit, static_argnums=(4, 5, 6))
def _edge_aggregate(h, srcp, dstp, zeros_nh, N, H, NB):
    npad = N + 8
    rpt = (N // NS) // 8 * 8
    tail0 = rpt * NS
    tail = N - tail0
    NP = NB // 2

    packed = (srcp | (dstp << 16)).reshape(NW * NB, K_EDGE)

    mesh = plsc.VectorSubcoreMesh(core_axis_name="c", subcore_axis_name="s")

    @functools.partial(
        pl.kernel,
        out_type=jax.ShapeDtypeStruct((NC, N, H), jnp.float32),
        mesh=mesh,
        scratch_types=[
            pltpu.VMEM((NB, K_EDGE), jnp.int32),
            pltpu.VMEM((K_EDGE,), jnp.int32),
            pltpu.VMEM((K_EDGE,), jnp.int32),
            pltpu.VMEM((K_EDGE,), jnp.int32),
            pltpu.VMEM((K_EDGE,), jnp.int32),
            pltpu.VMEM((K_EDGE, H), jnp.float32),
            pltpu.VMEM((K_EDGE, H), jnp.float32),
            pltpu.VMEM_SHARED((npad, H), jnp.float32),
            pltpu.SemaphoreType.DMA,
            pltpu.SemaphoreType.DMA,
            pltpu.SemaphoreType.DMA,
            pltpu.SemaphoreType.DMA,
        ],
    )
    def sc_kernel(h_hbm, pk_hbm, z_hbm, out_hbm, pidx, sidx0, didx0, sidx1,
                  didx1, rows0, rows1, aggr, sem0, sem1, sems0, sems1):
        c = lax.axis_index("c")
        s = lax.axis_index("s")
        wid = s * NC + c
        r0 = s * rpt
        pltpu.sync_copy(pk_hbm.at[pl.ds(wid * NB, NB)], pidx)
        pltpu.sync_copy(z_hbm.at[pl.ds(r0, rpt)], aggr.at[pl.ds(r0, rpt)])
        if tail > 0:
            @pl.when(s == NS - 1)
            def _():
                pltpu.sync_copy(z_hbm.at[pl.ds(tail0, tail)],
                                aggr.at[pl.ds(tail0, tail)])
        plsc.subcore_barrier()

        def unpack(j, sb, db):
            for i in range(K_EDGE // 16):
                v = pidx[j, pl.ds(i * 16, 16)]
                sb[pl.ds(i * 16, 16)] = v & 0xFFFF
                db[pl.ds(i * 16, 16)] = jnp.right_shift(v, 16)

        def gather(sb, rows, sem):
            return pltpu.make_async_copy(h_hbm.at[sb], rows, sem)

        unpack(0, sidx0, didx0)
        unpack(1, sidx1, didx1)
        gather(sidx0, rows0, sem0).start()
        gather(sidx1, rows1, sem1).start()

        def body(p, carry):
            j0 = 2 * p
            gather(sidx0, rows0, sem0).wait()
            pltpu.async_copy(rows0, aggr.at[didx0], sems0, add=True)
            gather(sidx1, rows1, sem1).wait()
            pltpu.async_copy(rows1, aggr.at[didx1], sems1, add=True)
            pltpu.make_async_copy(rows0, aggr.at[didx0], sems0).wait()
            unpack(j0 + 2, sidx0, didx0)
            gather(sidx0, rows0, sem0).start()
            pltpu.make_async_copy(rows1, aggr.at[didx1], sems1).wait()
            unpack(j0 + 3, sidx1, didx1)
            gather(sidx1, rows1, sem1).start()
            return carry

        lax.fori_loop(0, NP - 1, body, 0)
        gather(sidx0, rows0, sem0).wait()
        pltpu.async_copy(rows0, aggr.at[didx0], sems0, add=True)
        gather(sidx1, rows1, sem1).wait()
        pltpu.async_copy(rows1, aggr.at[didx1], sems1, add=True)
        pltpu.make_async_copy(rows0, aggr.at[didx0], sems0).wait()
        pltpu.make_async_copy(rows1, aggr.at[didx1], sems1).wait()

        plsc.subcore_barrier()
        pltpu.sync_copy(aggr.at[pl.ds(r0, rpt)], out_hbm.at[c, pl.ds(r0, rpt)])
        if tail > 0:
            @pl.when(s == NS - 1)
            def _():
                pltpu.sync_copy(aggr.at[pl.ds(tail0, tail)],
                                out_hbm.at[c, pl.ds(tail0, tail)])

    return sc_kernel(h, packed, zeros_nh)


@functools.partial(jax.jit, static_argnums=(3, 4, 5))
def _count_matrix(packed2d, batch1d, zeros_flat, N, G, NB):
    npad = N + 8
    NG = N * G
    wpt = NG // NS

    mesh = plsc.VectorSubcoreMesh(core_axis_name="c", subcore_axis_name="s")

    NP = NB // 2
    scratch_types = [pltpu.VMEM((NB, K_EDGE), jnp.int32)]
    scratch_types += [pltpu.VMEM((K_EDGE,), jnp.int32)] * 6
    scratch_types += [
        pltpu.VMEM((K_EDGE,), jnp.float32),
        pltpu.VMEM_SHARED((npad * G,), jnp.float32),
    ]
    scratch_types += [pltpu.SemaphoreType.DMA] * 4

    @functools.partial(
        pl.kernel,
        out_type=jax.ShapeDtypeStruct((NC, NG), jnp.float32),
        mesh=mesh,
        scratch_types=scratch_types,
    )
    def sc_kernel(pk_hbm, b_hbm, z_hbm, out_hbm, pidx, sbuf0, sbuf1, bbuf0,
                  bbuf1, fbuf0, fbuf1, ones, mflat, semb0, semb1, sems0,
                  sems1):
        c = lax.axis_index("c")
        s = lax.axis_index("s")
        wid = s * NC + c
        pltpu.sync_copy(pk_hbm.at[pl.ds(wid * NB, NB)], pidx)
        pltpu.sync_copy(z_hbm.at[pl.ds(s * wpt, wpt)],
                        mflat.at[pl.ds(s * wpt, wpt)])
        for i in range(K_EDGE // 16):
            ones[pl.ds(i * 16, 16)] = jnp.full((16,), 1.0, jnp.float32)
        plsc.subcore_barrier()

        def unpack_s(j, sb):
            for i in range(K_EDGE // 16):
                v = pidx[j, pl.ds(i * 16, 16)]
                sb[pl.ds(i * 16, 16)] = v & 0xFFFF

        def fill_f(j, bb, fb):
            for i in range(K_EDGE // 16):
                v = pidx[j, pl.ds(i * 16, 16)]
                fb[pl.ds(i * 16, 16)] = (jnp.right_shift(v, 16) * G
                                         + bb[pl.ds(i * 16, 16)])

        def bgather(sb, bb, sem):
            return pltpu.make_async_copy(b_hbm.at[sb], bb, sem)

        def scat(fb, sem):
            return pltpu.make_async_copy(ones, mflat.at[fb], sem)

        unpack_s(0, sbuf0)
        bgather(sbuf0, bbuf0, semb0).start()
        unpack_s(1, sbuf1)
        bgather(sbuf1, bbuf1, semb1).start()

        def body(p, carry):
            j0 = 2 * p
            bgather(sbuf0, bbuf0, semb0).wait()
            fill_f(j0, bbuf0, fbuf0)
            pltpu.async_copy(ones, mflat.at[fbuf0], sems0, add=True)
            bgather(sbuf1, bbuf1, semb1).wait()
            fill_f(j0 + 1, bbuf1, fbuf1)
            pltpu.async_copy(ones, mflat.at[fbuf1], sems1, add=True)
            scat(fbuf0, sems0).wait()
            unpack_s(j0 + 2, sbuf0)
            bgather(sbuf0, bbuf0, semb0).start()
            scat(fbuf1, sems1).wait()
            unpack_s(j0 + 3, sbuf1)
            bgather(sbuf1, bbuf1, semb1).start()
            return carry

        lax.fori_loop(0, NP - 1, body, 0)
        j0 = NB - 2
        bgather(sbuf0, bbuf0, semb0).wait()
        fill_f(j0, bbuf0, fbuf0)
        pltpu.async_copy(ones, mflat.at[fbuf0], sems0, add=True)
        bgather(sbuf1, bbuf1, semb1).wait()
        fill_f(j0 + 1, bbuf1, fbuf1)
        pltpu.async_copy(ones, mflat.at[fbuf1], sems1, add=True)
        scat(fbuf0, sems0).wait()
        scat(fbuf1, sems1).wait()
        plsc.subcore_barrier()
        pltpu.sync_copy(mflat.at[pl.ds(s * wpt, wpt)],
                        out_hbm.at[c, pl.ds(s * wpt, wpt)])

    return sc_kernel(packed2d, batch1d, zeros_flat)


def _onehot(b2d, bn, g):
    return (b2d == lax.broadcasted_iota(jnp.int32, (bn, g), 1)).astype(
        jnp.float32)


def _msum_body(a_ref, b_ref, o_ref):
    o_ref[...] = a_ref[...] + b_ref[...]


def _msum(a, b, BN):
    N, G = a.shape
    return pl.pallas_call(
        _msum_body,
        grid=(N // BN,),
        in_specs=[
            pl.BlockSpec((BN, G), lambda i: (i, 0)),
            pl.BlockSpec((BN, G), lambda i: (i, 0)),
        ],
        out_specs=pl.BlockSpec((BN, G), lambda i: (i, 0)),
        out_shape=jax.ShapeDtypeStruct((N, G), jnp.float32),
    )(a, b)


def _gin_common(bn, g, h_ref, p0_ref, p1_ref, m_ref, vn_ref, b_ref, w1_ref,
                b1_ref, w2_ref, b2_ref, eps_ref):
    vn = vn_ref[...]
    oh = _onehot(b_ref[...], bn, g)
    h_in = h_ref[...] + jnp.dot(oh, vn, preferred_element_type=jnp.float32)
    aggr = (p0_ref[...] + p1_ref[...]
            + jnp.dot(m_ref[...], vn, preferred_element_type=jnp.float32))
    z = (1.0 + eps_ref[0, 0]) * h_in + aggr
    hid = jnp.maximum(
        jnp.dot(z, w1_ref[...], preferred_element_type=jnp.float32)
        + b1_ref[...], 0.0)
    z2 = jnp.dot(hid, w2_ref[...],
                 preferred_element_type=jnp.float32) + b2_ref[...]
    h_new = jnp.maximum(z2, 0.0) + h_in
    return h_in, h_new, oh


def _dense_body(bn, g, h_ref, p0_ref, p1_ref, m_ref, vn_ref, b_ref, w1_ref,
                b1_ref, w2_ref, b2_ref, eps_ref, hn_ref, seg_ref):
    h_in, h_new, oh = _gin_common(bn, g, h_ref, p0_ref, p1_ref, m_ref, vn_ref,
                                  b_ref, w1_ref, b1_ref, w2_ref, b2_ref,
                                  eps_ref)
    hn_ref[...] = h_new
    seg = jnp.dot(oh.T, h_in, preferred_element_type=jnp.float32)

    @pl.when(pl.program_id(0) == 0)
    def _():
        seg_ref[...] = seg

    @pl.when(pl.program_id(0) != 0)
    def _():
        seg_ref[...] += seg


def _gin_in_specs(BN, H, H2, G):
    return [
        pl.BlockSpec((BN, H), lambda i: (i, 0)),
        pl.BlockSpec((BN, H), lambda i: (i, 0)),
        pl.BlockSpec((BN, H), lambda i: (i, 0)),
        pl.BlockSpec((BN, G), lambda i: (i, 0)),
        pl.BlockSpec((G, H), lambda i: (0, 0)),
        pl.BlockSpec((BN, 1), lambda i: (i, 0)),
        pl.BlockSpec((H, H2), lambda i: (0, 0)),
        pl.BlockSpec((H2,), lambda i: (0,)),
        pl.BlockSpec((H2, H), lambda i: (0, 0)),
        pl.BlockSpec((H,), lambda i: (0,)),
        pl.BlockSpec((1, 1), lambda i: (0, 0)),
    ]


def _gin_dense(h, p0, p1, m, vn, b2d, w1, b1, w2, b2, epsl, G, BN):
    N, H = h.shape
    H2 = w1.shape[1]
    return pl.pallas_call(
        functools.partial(_dense_body, BN, G),
        grid=(N // BN,),
        in_specs=_gin_in_specs(BN, H, H2, G),
        out_specs=[
            pl.BlockSpec((BN, H), lambda i: (i, 0)),
            pl.BlockSpec((G, H), lambda i: (0, 0)),
        ],
        out_shape=[
            jax.ShapeDtypeStruct((N, H), jnp.float32),
            jax.ShapeDtypeStruct((G, H), jnp.float32),
        ],
    )(h, p0, p1, m, vn, b2d, w1, b1, w2, b2, epsl)


def _last_body(bn, g, h_ref, p0_ref, p1_ref, m_ref, vn_ref, b_ref, w1_ref,
               b1_ref, w2_ref, b2_ref, eps_ref, segmax_ref):
    _, h_new, _ = _gin_common(bn, g, h_ref, p0_ref, p1_ref, m_ref, vn_ref,
                              b_ref, w1_ref, b1_ref, w2_ref, b2_ref, eps_ref)
    b2d = b_ref[...]
    first = pl.program_id(0) == 0

    def body(gi, carry):
        vals = jnp.where(b2d == gi, h_new, -jnp.inf)
        row = jnp.max(vals, axis=0, keepdims=True)
        cur = segmax_ref[pl.ds(gi, 1), :]
        segmax_ref[pl.ds(gi, 1), :] = jnp.where(first, row,
                                                jnp.maximum(cur, row))
        return carry

    lax.fori_loop(0, g, body, 0)


def _gin_last(h, p0, p1, m, vn, b2d, w1, b1, w2, b2, epsl, G, BN):
    N, H = h.shape
    H2 = w1.shape[1]
    return pl.pallas_call(
        functools.partial(_last_body, BN, G),
        grid=(N // BN,),
        in_specs=_gin_in_specs(BN, H, H2, G),
        out_specs=pl.BlockSpec((G, H), lambda i: (0, 0)),
        out_shape=jax.ShapeDtypeStruct((G, H), jnp.float32),
    )(h, p0, p1, m, vn, b2d, w1, b1, w2, b2, epsl)


def _vn_body(seg_ref, vn_ref, w1_ref, b1_ref, w2_ref, b2_ref, out_ref):
    vn = vn_ref[...]
    t = seg_ref[...] + vn
    t = jnp.maximum(
        jnp.dot(t, w1_ref[...], preferred_element_type=jnp.float32)
        + b1_ref[...], 0.0)
    out_ref[...] = vn + jnp.maximum(
        jnp.dot(t, w2_ref[...], preferred_element_type=jnp.float32)
        + b2_ref[...], 0.0)


def _vn_update(seg, vn, w1, b1, w2, b2):
    return pl.pallas_call(
        _vn_body,
        out_shape=jax.ShapeDtypeStruct(vn.shape, jnp.float32),
    )(seg, vn, w1, b1, w2, b2)


def _pred_body(hrep_ref, mg_ref, mc_ref, wa_ref, wb_ref, wc_ref, bp1_ref,
               wp2_ref, bp2_ref, out_ref):
    acc = (jnp.dot(hrep_ref[...], wa_ref[...],
                   preferred_element_type=jnp.float32)
           + jnp.dot(mg_ref[...], wb_ref[...],
                     preferred_element_type=jnp.float32)
           + jnp.dot(mc_ref[...], wc_ref[...],
                     preferred_element_type=jnp.float32)
           + bp1_ref[...])
    out_ref[...] = jnp.dot(jnp.maximum(acc, 0.0), wp2_ref[...],
                           preferred_element_type=jnp.float32) + bp2_ref[...]


def _predictor(hrep, morgan, maccs, wpa, wpb, wpc, bp1, wp2, bp2):
    G = hrep.shape[0]
    T = wp2.shape[1]
    return pl.pallas_call(
        _pred_body,
        out_shape=jax.ShapeDtypeStruct((G, T), jnp.float32),
    )(hrep, morgan, maccs, wpa, wpb, wpc, bp1, wp2, bp2)


def kernel(x, edge_index, batch, morgan, maccs, W1, b1, W2, b2, eps,
           vnW1, vnb1, vnW2, vnb2, Wp1, bp1, Wp2, bp2):
    N, H = x.shape
    E = edge_index.shape[1]
    G = morgan.shape[0]
    L = W1.shape[0]
    BN = 2000

    src = edge_index[0].astype(jnp.int32)
    dst = edge_index[1].astype(jnp.int32)
    b2d = batch.astype(jnp.int32).reshape(N, 1)

    per_tile = -(-E // NW)
    NB = -(-per_tile // K_EDGE)
    NB = -(-NB // 8) * 8
    E_pad = NW * NB * K_EDGE
    pad = E_pad - E
    pr = jnp.arange(pad, dtype=jnp.int32) % 8
    srcp = jnp.concatenate([src, pr])
    dstp = jnp.concatenate([dst, N + pr])
    zeros_nh = jnp.zeros((N, H), jnp.float32)
    zeros_ng = jnp.zeros((N * G,), jnp.float32)
    packed = (srcp | (dstp << 16)).reshape(NW * NB, K_EDGE)

    mp = _count_matrix(packed, batch.astype(jnp.int32), zeros_ng, N, G, NB)
    m = _msum(mp[0].reshape(N, G), mp[1].reshape(N, G), BN)

    vn = jnp.zeros((G, H), jnp.float32)
    h = x
    for l in range(L):
        parts = _edge_aggregate(h, srcp, dstp, zeros_nh, N, H, NB)
        eps_l = eps[l].reshape(1, 1)
        if l < L - 1:
            h, seg = _gin_dense(h, parts[0], parts[1], m, vn, b2d, W1[l],
                                b1[l], W2[l], b2[l], eps_l, G, BN)
            vn = _vn_update(seg, vn, vnW1[l], vnb1[l], vnW2[l], vnb2[l])
        else:
            hrep = _gin_last(h, parts[0], parts[1], m, vn, b2d, W1[l], b1[l],
                             W2[l], b2[l], eps_l, G, BN)

    return _predictor(hrep, morgan, maccs, Wp1[:H], Wp1[H:H + 1024],
                      Wp1[H + 1024:], bp1, Wp2, bp2)


# speedup vs baseline: 1.0570x; 1.0570x over previous
import functools

import jax
import jax.numpy as jnp
from jax import lax
from jax.experimental import pallas as pl
from jax.experimental.pallas import tpu as pltpu
from jax.experimental.pallas import tpu_sc as plsc

NC = 2
NS = 16
NW = NC * NS
K_EDGE = 128


@functools.partial(jax.jit, static_argnums=(4, 5, 6))
def _edge_aggregate(h, srcp, dstp, zeros_nh, N, H, NB):
    npad = N + 8
    rpt = (N // NS) // 8 * 8
    tail0 = rpt * NS
    tail = N - tail0
    NP = NB // 2

    packed = (srcp | (dstp << 16)).reshape(NW * NB, K_EDGE)

    mesh = plsc.VectorSubcoreMesh(core_axis_name="c", subcore_axis_name="s")

    @functools.partial(
        pl.kernel,
        out_type=jax.ShapeDtypeStruct((NC, N, H), jnp.float32),
        mesh=mesh,
        scratch_types=[
            pltpu.VMEM((NB, K_EDGE), jnp.int32),
            pltpu.VMEM((K_EDGE,), jnp.int32),
            pltpu.VMEM((K_EDGE,), jnp.int32),
            pltpu.VMEM((K_EDGE,), jnp.int32),
            pltpu.VMEM((K_EDGE,), jnp.int32),
            pltpu.VMEM((K_EDGE, H), jnp.float32),
            pltpu.VMEM((K_EDGE, H), jnp.float32),
            pltpu.VMEM_SHARED((npad, H), jnp.float32),
            pltpu.SemaphoreType.DMA,
            pltpu.SemaphoreType.DMA,
            pltpu.SemaphoreType.DMA,
            pltpu.SemaphoreType.DMA,
        ],
    )
    def sc_kernel(h_hbm, pk_hbm, z_hbm, out_hbm, pidx, sidx0, didx0, sidx1,
                  didx1, rows0, rows1, aggr, sem0, sem1, sems0, sems1):
        c = lax.axis_index("c")
        s = lax.axis_index("s")
        wid = s * NC + c
        r0 = s * rpt
        pltpu.sync_copy(pk_hbm.at[pl.ds(wid * NB, NB)], pidx)
        pltpu.sync_copy(z_hbm.at[pl.ds(r0, rpt)], aggr.at[pl.ds(r0, rpt)])
        if tail > 0:
            @pl.when(s == NS - 1)
            def _():
                pltpu.sync_copy(z_hbm.at[pl.ds(tail0, tail)],
                                aggr.at[pl.ds(tail0, tail)])
        plsc.subcore_barrier()

        def unpack(j, sb, db):
            for i in range(K_EDGE // 16):
                v = pidx[j, pl.ds(i * 16, 16)]
                sb[pl.ds(i * 16, 16)] = v & 0xFFFF
                db[pl.ds(i * 16, 16)] = jnp.right_shift(v, 16)

        def gather(sb, rows, sem):
            return pltpu.make_async_copy(h_hbm.at[sb], rows, sem)

        unpack(0, sidx0, didx0)
        unpack(1, sidx1, didx1)
        gather(sidx0, rows0, sem0).start()
        gather(sidx1, rows1, sem1).start()

        def body(p, carry):
            j0 = 2 * p
            gather(sidx0, rows0, sem0).wait()
            pltpu.sync_copy(rows0, aggr.at[didx0], add=True)
            unpack(j0 + 2, sidx0, didx0)
            gather(sidx0, rows0, sem0).start()
            gather(sidx1, rows1, sem1).wait()
            pltpu.sync_copy(rows1, aggr.at[didx1], add=True)
            unpack(j0 + 3, sidx1, didx1)
            gather(sidx1, rows1, sem1).start()
            return carry

        lax.fori_loop(0, NP - 1, body, 0)
        gather(sidx0, rows0, sem0).wait()
        pltpu.sync_copy(rows0, aggr.at[didx0], add=True)
        gather(sidx1, rows1, sem1).wait()
        pltpu.sync_copy(rows1, aggr.at[didx1], add=True)

        plsc.subcore_barrier()
        pltpu.sync_copy(aggr.at[pl.ds(r0, rpt)], out_hbm.at[c, pl.ds(r0, rpt)])
        if tail > 0:
            @pl.when(s == NS - 1)
            def _():
                pltpu.sync_copy(aggr.at[pl.ds(tail0, tail)],
                                out_hbm.at[c, pl.ds(tail0, tail)])

    return sc_kernel(h, packed, zeros_nh)


@functools.partial(jax.jit, static_argnums=(3, 4, 5))
def _count_matrix(packed2d, batch1d, zeros_flat, N, G, NB):
    npad = N + 8
    NG = N * G
    wpt = NG // NS

    mesh = plsc.VectorSubcoreMesh(core_axis_name="c", subcore_axis_name="s")

    NP = NB // 2
    scratch_types = [pltpu.VMEM((NB, K_EDGE), jnp.int32)]
    scratch_types += [pltpu.VMEM((K_EDGE,), jnp.int32)] * 6
    scratch_types += [
        pltpu.VMEM((K_EDGE,), jnp.float32),
        pltpu.VMEM_SHARED((npad * G,), jnp.float32),
    ]
    scratch_types += [pltpu.SemaphoreType.DMA] * 4

    @functools.partial(
        pl.kernel,
        out_type=jax.ShapeDtypeStruct((NC, NG), jnp.float32),
        mesh=mesh,
        scratch_types=scratch_types,
    )
    def sc_kernel(pk_hbm, b_hbm, z_hbm, out_hbm, pidx, sbuf0, sbuf1, bbuf0,
                  bbuf1, fbuf0, fbuf1, ones, mflat, semb0, semb1, sems0,
                  sems1):
        c = lax.axis_index("c")
        s = lax.axis_index("s")
        wid = s * NC + c
        pltpu.sync_copy(pk_hbm.at[pl.ds(wid * NB, NB)], pidx)
        pltpu.sync_copy(z_hbm.at[pl.ds(s * wpt, wpt)],
                        mflat.at[pl.ds(s * wpt, wpt)])
        for i in range(K_EDGE // 16):
            ones[pl.ds(i * 16, 16)] = jnp.full((16,), 1.0, jnp.float32)
        plsc.subcore_barrier()

        def unpack_s(j, sb):
            for i in range(K_EDGE // 16):
                v = pidx[j, pl.ds(i * 16, 16)]
                sb[pl.ds(i * 16, 16)] = v & 0xFFFF

        def fill_f(j, bb, fb):
            for i in range(K_EDGE // 16):
                v = pidx[j, pl.ds(i * 16, 16)]
                fb[pl.ds(i * 16, 16)] = (jnp.right_shift(v, 16) * G
                                         + bb[pl.ds(i * 16, 16)])

        def bgather(sb, bb, sem):
            return pltpu.make_async_copy(b_hbm.at[sb], bb, sem)

        def scat(fb, sem):
            return pltpu.make_async_copy(ones, mflat.at[fb], sem)

        unpack_s(0, sbuf0)
        bgather(sbuf0, bbuf0, semb0).start()
        unpack_s(1, sbuf1)
        bgather(sbuf1, bbuf1, semb1).start()

        def body(p, carry):
            j0 = 2 * p
            bgather(sbuf0, bbuf0, semb0).wait()
            fill_f(j0, bbuf0, fbuf0)
            pltpu.async_copy(ones, mflat.at[fbuf0], sems0, add=True)
            bgather(sbuf1, bbuf1, semb1).wait()
            fill_f(j0 + 1, bbuf1, fbuf1)
            pltpu.async_copy(ones, mflat.at[fbuf1], sems1, add=True)
            scat(fbuf0, sems0).wait()
            unpack_s(j0 + 2, sbuf0)
            bgather(sbuf0, bbuf0, semb0).start()
            scat(fbuf1, sems1).wait()
            unpack_s(j0 + 3, sbuf1)
            bgather(sbuf1, bbuf1, semb1).start()
            return carry

        lax.fori_loop(0, NP - 1, body, 0)
        j0 = NB - 2
        bgather(sbuf0, bbuf0, semb0).wait()
        fill_f(j0, bbuf0, fbuf0)
        pltpu.async_copy(ones, mflat.at[fbuf0], sems0, add=True)
        bgather(sbuf1, bbuf1, semb1).wait()
        fill_f(j0 + 1, bbuf1, fbuf1)
        pltpu.async_copy(ones, mflat.at[fbuf1], sems1, add=True)
        scat(fbuf0, sems0).wait()
        scat(fbuf1, sems1).wait()
        plsc.subcore_barrier()
        pltpu.sync_copy(mflat.at[pl.ds(s * wpt, wpt)],
                        out_hbm.at[c, pl.ds(s * wpt, wpt)])

    return sc_kernel(packed2d, batch1d, zeros_flat)


def _onehot(b2d, bn, g):
    return (b2d == lax.broadcasted_iota(jnp.int32, (bn, g), 1)).astype(
        jnp.float32)


def _msum_body(a_ref, b_ref, o_ref):
    o_ref[...] = a_ref[...] + b_ref[...]


def _msum(a, b, BN):
    N, G = a.shape
    return pl.pallas_call(
        _msum_body,
        grid=(N // BN,),
        in_specs=[
            pl.BlockSpec((BN, G), lambda i: (i, 0)),
            pl.BlockSpec((BN, G), lambda i: (i, 0)),
        ],
        out_specs=pl.BlockSpec((BN, G), lambda i: (i, 0)),
        out_shape=jax.ShapeDtypeStruct((N, G), jnp.float32),
    )(a, b)


def _gin_common(bn, g, h_ref, p0_ref, p1_ref, m_ref, vn_ref, b_ref, w1_ref,
                b1_ref, w2_ref, b2_ref, eps_ref):
    vn = vn_ref[...]
    oh = _onehot(b_ref[...], bn, g)
    h_in = h_ref[...] + jnp.dot(oh, vn, preferred_element_type=jnp.float32)
    aggr = (p0_ref[...] + p1_ref[...]
            + jnp.dot(m_ref[...], vn, preferred_element_type=jnp.float32))
    z = (1.0 + eps_ref[0, 0]) * h_in + aggr
    hid = jnp.maximum(
        jnp.dot(z, w1_ref[...], preferred_element_type=jnp.float32)
        + b1_ref[...], 0.0)
    z2 = jnp.dot(hid, w2_ref[...],
                 preferred_element_type=jnp.float32) + b2_ref[...]
    h_new = jnp.maximum(z2, 0.0) + h_in
    return h_in, h_new, oh


def _dense_body(bn, g, h_ref, p0_ref, p1_ref, m_ref, vn_ref, b_ref, w1_ref,
                b1_ref, w2_ref, b2_ref, eps_ref, hn_ref, seg_ref):
    h_in, h_new, oh = _gin_common(bn, g, h_ref, p0_ref, p1_ref, m_ref, vn_ref,
                                  b_ref, w1_ref, b1_ref, w2_ref, b2_ref,
                                  eps_ref)
    hn_ref[...] = h_new
    seg = jnp.dot(oh.T, h_in, preferred_element_type=jnp.float32)

    @pl.when(pl.program_id(0) == 0)
    def _():
        seg_ref[...] = seg

    @pl.when(pl.program_id(0) != 0)
    def _():
        seg_ref[...] += seg


def _gin_in_specs(BN, H, H2, G):
    return [
        pl.BlockSpec((BN, H), lambda i: (i, 0)),
        pl.BlockSpec((BN, H), lambda i: (i, 0)),
        pl.BlockSpec((BN, H), lambda i: (i, 0)),
        pl.BlockSpec((BN, G), lambda i: (i, 0)),
        pl.BlockSpec((G, H), lambda i: (0, 0)),
        pl.BlockSpec((BN, 1), lambda i: (i, 0)),
        pl.BlockSpec((H, H2), lambda i: (0, 0)),
        pl.BlockSpec((H2,), lambda i: (0,)),
        pl.BlockSpec((H2, H), lambda i: (0, 0)),
        pl.BlockSpec((H,), lambda i: (0,)),
        pl.BlockSpec((1, 1), lambda i: (0, 0)),
    ]


def _gin_dense(h, p0, p1, m, vn, b2d, w1, b1, w2, b2, epsl, G, BN):
    N, H = h.shape
    H2 = w1.shape[1]
    return pl.pallas_call(
        functools.partial(_dense_body, BN, G),
        grid=(N // BN,),
        in_specs=_gin_in_specs(BN, H, H2, G),
        out_specs=[
            pl.BlockSpec((BN, H), lambda i: (i, 0)),
            pl.BlockSpec((G, H), lambda i: (0, 0)),
        ],
        out_shape=[
            jax.ShapeDtypeStruct((N, H), jnp.float32),
            jax.ShapeDtypeStruct((G, H), jnp.float32),
        ],
    )(h, p0, p1, m, vn, b2d, w1, b1, w2, b2, epsl)


def _last_body(bn, g, h_ref, p0_ref, p1_ref, m_ref, vn_ref, b_ref, w1_ref,
               b1_ref, w2_ref, b2_ref, eps_ref, segmax_ref):
    _, h_new, _ = _gin_common(bn, g, h_ref, p0_ref, p1_ref, m_ref, vn_ref,
                              b_ref, w1_ref, b1_ref, w2_ref, b2_ref, eps_ref)
    b2d = b_ref[...]
    first = pl.program_id(0) == 0

    def body(gi, carry):
        vals = jnp.where(b2d == gi, h_new, -jnp.inf)
        row = jnp.max(vals, axis=0, keepdims=True)
        cur = segmax_ref[pl.ds(gi, 1), :]
        segmax_ref[pl.ds(gi, 1), :] = jnp.where(first, row,
                                                jnp.maximum(cur, row))
        return carry

    lax.fori_loop(0, g, body, 0)


def _gin_last(h, p0, p1, m, vn, b2d, w1, b1, w2, b2, epsl, G, BN):
    N, H = h.shape
    H2 = w1.shape[1]
    return pl.pallas_call(
        functools.partial(_last_body, BN, G),
        grid=(N // BN,),
        in_specs=_gin_in_specs(BN, H, H2, G),
        out_specs=pl.BlockSpec((G, H), lambda i: (0, 0)),
        out_shape=jax.ShapeDtypeStruct((G, H), jnp.float32),
    )(h, p0, p1, m, vn, b2d, w1, b1, w2, b2, epsl)


def _vn_body(seg_ref, vn_ref, w1_ref, b1_ref, w2_ref, b2_ref, out_ref):
    vn = vn_ref[...]
    t = seg_ref[...] + vn
    t = jnp.maximum(
        jnp.dot(t, w1_ref[...], preferred_element_type=jnp.float32)
        + b1_ref[...], 0.0)
    out_ref[...] = vn + jnp.maximum(
        jnp.dot(t, w2_ref[...], preferred_element_type=jnp.float32)
        + b2_ref[...], 0.0)


def _vn_update(seg, vn, w1, b1, w2, b2):
    return pl.pallas_call(
        _vn_body,
        out_shape=jax.ShapeDtypeStruct(vn.shape, jnp.float32),
    )(seg, vn, w1, b1, w2, b2)


def _pred_body(hrep_ref, mg_ref, mc_ref, wa_ref, wb_ref, wc_ref, bp1_ref,
               wp2_ref, bp2_ref, out_ref):
    acc = (jnp.dot(hrep_ref[...], wa_ref[...],
                   preferred_element_type=jnp.float32)
           + jnp.dot(mg_ref[...], wb_ref[...],
                     preferred_element_type=jnp.float32)
           + jnp.dot(mc_ref[...], wc_ref[...],
                     preferred_element_type=jnp.float32)
           + bp1_ref[...])
    out_ref[...] = jnp.dot(jnp.maximum(acc, 0.0), wp2_ref[...],
                           preferred_element_type=jnp.float32) + bp2_ref[...]


def _predictor(hrep, morgan, maccs, wpa, wpb, wpc, bp1, wp2, bp2):
    G = hrep.shape[0]
    T = wp2.shape[1]
    return pl.pallas_call(
        _pred_body,
        out_shape=jax.ShapeDtypeStruct((G, T), jnp.float32),
    )(hrep, morgan, maccs, wpa, wpb, wpc, bp1, wp2, bp2)


def kernel(x, edge_index, batch, morgan, maccs, W1, b1, W2, b2, eps,
           vnW1, vnb1, vnW2, vnb2, Wp1, bp1, Wp2, bp2):
    N, H = x.shape
    E = edge_index.shape[1]
    G = morgan.shape[0]
    L = W1.shape[0]
    BN = 2000

    src = edge_index[0].astype(jnp.int32)
    dst = edge_index[1].astype(jnp.int32)
    b2d = batch.astype(jnp.int32).reshape(N, 1)

    per_tile = -(-E // NW)
    NB = -(-per_tile // K_EDGE)
    NB = -(-NB // 8) * 8
    E_pad = NW * NB * K_EDGE
    pad = E_pad - E
    pr = jnp.arange(pad, dtype=jnp.int32) % 8
    srcp = jnp.concatenate([src, pr])
    dstp = jnp.concatenate([dst, N + pr])
    zeros_nh = jnp.zeros((N, H), jnp.float32)
    zeros_ng = jnp.zeros((N * G,), jnp.float32)
    packed = (srcp | (dstp << 16)).reshape(NW * NB, K_EDGE)

    mp = _count_matrix(packed, batch.astype(jnp.int32), zeros_ng, N, G, NB)
    m = _msum(mp[0].reshape(N, G), mp[1].reshape(N, G), BN)

    vn = jnp.zeros((G, H), jnp.float32)
    h = x
    for l in range(L):
        parts = _edge_aggregate(h, srcp, dstp, zeros_nh, N, H, NB)
        eps_l = eps[l].reshape(1, 1)
        if l < L - 1:
            h, seg = _gin_dense(h, parts[0], parts[1], m, vn, b2d, W1[l],
                                b1[l], W2[l], b2[l], eps_l, G, BN)
            vn = _vn_update(seg, vn, vnW1[l], vnb1[l], vnW2[l], vnb2[l])
        else:
            hrep = _gin_last(h, parts[0], parts[1], m, vn, b2d, W1[l], b1[l],
                             W2[l], b2[l], eps_l, G, BN)

    return _predictor(hrep, morgan, maccs, Wp1[:H], Wp1[H:H + 1024],
                      Wp1[H + 1024:], bp1, Wp2, bp2)


# speedup vs baseline: 1.1371x; 1.0758x over previous
import functools

import jax
import jax.numpy as jnp
from jax import lax
from jax.experimental import pallas as pl
from jax.experimental.pallas import tpu as pltpu
from jax.experimental.pallas import tpu_sc as plsc

NC = 2
NS = 16
NW = NC * NS
K_EDGE = 128


@functools.partial(jax.jit, static_argnums=(4, 5, 6))
def _edge_aggregate(h, srcp, dstp, zeros_nh, N, H, NB):
    npad = N + 8
    rpt = (N // NS) // 8 * 8
    tail0 = rpt * NS
    tail = N - tail0
    NP = NB // 2

    packed = (srcp | (dstp << 16)).reshape(NW * NB, K_EDGE)

    mesh = plsc.VectorSubcoreMesh(core_axis_name="c", subcore_axis_name="s")

    @functools.partial(
        pl.kernel,
        out_type=jax.ShapeDtypeStruct((NC, N, H), jnp.float32),
        mesh=mesh,
        scratch_types=[
            pltpu.VMEM((NB, K_EDGE), jnp.int32),
            pltpu.VMEM((K_EDGE,), jnp.int32),
            pltpu.VMEM((K_EDGE,), jnp.int32),
            pltpu.VMEM((K_EDGE,), jnp.int32),
            pltpu.VMEM((K_EDGE,), jnp.int32),
            pltpu.VMEM((K_EDGE, H), jnp.float32),
            pltpu.VMEM((K_EDGE, H), jnp.float32),
            pltpu.VMEM_SHARED((npad, H), jnp.float32),
            pltpu.SemaphoreType.DMA,
            pltpu.SemaphoreType.DMA,
            pltpu.SemaphoreType.DMA,
            pltpu.SemaphoreType.DMA,
        ],
    )
    def sc_kernel(h_hbm, pk_hbm, z_hbm, out_hbm, pidx, sidx0, didx0, sidx1,
                  didx1, rows0, rows1, aggr, sem0, sem1, sems0, sems1):
        c = lax.axis_index("c")
        s = lax.axis_index("s")
        wid = s * NC + c
        r0 = s * rpt
        pltpu.sync_copy(pk_hbm.at[pl.ds(wid * NB, NB)], pidx)
        pltpu.sync_copy(z_hbm.at[pl.ds(r0, rpt)], aggr.at[pl.ds(r0, rpt)])
        if tail > 0:
            @pl.when(s == NS - 1)
            def _():
                pltpu.sync_copy(z_hbm.at[pl.ds(tail0, tail)],
                                aggr.at[pl.ds(tail0, tail)])
        plsc.subcore_barrier()

        def unpack(j, sb, db):
            for i in range(K_EDGE // 16):
                v = pidx[j, pl.ds(i * 16, 16)]
                sb[pl.ds(i * 16, 16)] = v & 0xFFFF
                db[pl.ds(i * 16, 16)] = jnp.right_shift(v, 16)

        def gather(sb, rows, sem):
            return pltpu.make_async_copy(h_hbm.at[sb], rows, sem)

        unpack(0, sidx0, didx0)
        unpack(1, sidx1, didx1)
        gather(sidx0, rows0, sem0).start()
        gather(sidx1, rows1, sem1).start()

        def body(p, carry):
            j0 = 2 * p
            gather(sidx0, rows0, sem0).wait()
            pltpu.sync_copy(rows0, aggr.at[didx0], add=True)
            unpack(j0 + 2, sidx0, didx0)
            gather(sidx0, rows0, sem0).start()
            gather(sidx1, rows1, sem1).wait()
            pltpu.sync_copy(rows1, aggr.at[didx1], add=True)
            unpack(j0 + 3, sidx1, didx1)
            gather(sidx1, rows1, sem1).start()
            return carry

        lax.fori_loop(0, NP - 1, body, 0)
        gather(sidx0, rows0, sem0).wait()
        pltpu.sync_copy(rows0, aggr.at[didx0], add=True)
        gather(sidx1, rows1, sem1).wait()
        pltpu.sync_copy(rows1, aggr.at[didx1], add=True)

        plsc.subcore_barrier()
        pltpu.sync_copy(aggr.at[pl.ds(r0, rpt)], out_hbm.at[c, pl.ds(r0, rpt)])
        if tail > 0:
            @pl.when(s == NS - 1)
            def _():
                pltpu.sync_copy(aggr.at[pl.ds(tail0, tail)],
                                out_hbm.at[c, pl.ds(tail0, tail)])

    return sc_kernel(h, packed, zeros_nh)


@functools.partial(jax.jit, static_argnums=(3, 4, 5))
def _count_matrix(packed2d, batch1d, zeros_flat, N, G, NB):
    npad = N + 8
    NG = N * G
    wpt = NG // NS

    mesh = plsc.VectorSubcoreMesh(core_axis_name="c", subcore_axis_name="s")

    NP = NB // 2
    scratch_types = [pltpu.VMEM((NB, K_EDGE), jnp.int32)]
    scratch_types += [pltpu.VMEM((K_EDGE,), jnp.int32)] * 6
    scratch_types += [
        pltpu.VMEM((K_EDGE,), jnp.float32),
        pltpu.VMEM_SHARED((npad * G,), jnp.float32),
    ]
    scratch_types += [pltpu.SemaphoreType.DMA] * 4

    @functools.partial(
        pl.kernel,
        out_type=jax.ShapeDtypeStruct((NC, NG), jnp.float32),
        mesh=mesh,
        scratch_types=scratch_types,
    )
    def sc_kernel(pk_hbm, b_hbm, z_hbm, out_hbm, pidx, sbuf0, sbuf1, bbuf0,
                  bbuf1, fbuf0, fbuf1, ones, mflat, semb0, semb1, sems0,
                  sems1):
        c = lax.axis_index("c")
        s = lax.axis_index("s")
        wid = s * NC + c
        pltpu.sync_copy(pk_hbm.at[pl.ds(wid * NB, NB)], pidx)
        pltpu.sync_copy(z_hbm.at[pl.ds(s * wpt, wpt)],
                        mflat.at[pl.ds(s * wpt, wpt)])
        for i in range(K_EDGE // 16):
            ones[pl.ds(i * 16, 16)] = jnp.full((16,), 1.0, jnp.float32)
        plsc.subcore_barrier()

        def unpack_s(j, sb):
            for i in range(K_EDGE // 16):
                v = pidx[j, pl.ds(i * 16, 16)]
                sb[pl.ds(i * 16, 16)] = v & 0xFFFF

        def fill_f(j, bb, fb):
            for i in range(K_EDGE // 16):
                v = pidx[j, pl.ds(i * 16, 16)]
                fb[pl.ds(i * 16, 16)] = (jnp.right_shift(v, 16) * G
                                         + bb[pl.ds(i * 16, 16)])

        def bgather(sb, bb, sem):
            return pltpu.make_async_copy(b_hbm.at[sb], bb, sem)

        def scat(fb, sem):
            return pltpu.make_async_copy(ones, mflat.at[fb], sem)

        unpack_s(0, sbuf0)
        bgather(sbuf0, bbuf0, semb0).start()
        unpack_s(1, sbuf1)
        bgather(sbuf1, bbuf1, semb1).start()

        def body(p, carry):
            j0 = 2 * p
            bgather(sbuf0, bbuf0, semb0).wait()
            fill_f(j0, bbuf0, fbuf0)
            pltpu.async_copy(ones, mflat.at[fbuf0], sems0, add=True)
            bgather(sbuf1, bbuf1, semb1).wait()
            fill_f(j0 + 1, bbuf1, fbuf1)
            pltpu.async_copy(ones, mflat.at[fbuf1], sems1, add=True)
            scat(fbuf0, sems0).wait()
            unpack_s(j0 + 2, sbuf0)
            bgather(sbuf0, bbuf0, semb0).start()
            scat(fbuf1, sems1).wait()
            unpack_s(j0 + 3, sbuf1)
            bgather(sbuf1, bbuf1, semb1).start()
            return carry

        lax.fori_loop(0, NP - 1, body, 0)
        j0 = NB - 2
        bgather(sbuf0, bbuf0, semb0).wait()
        fill_f(j0, bbuf0, fbuf0)
        pltpu.async_copy(ones, mflat.at[fbuf0], sems0, add=True)
        bgather(sbuf1, bbuf1, semb1).wait()
        fill_f(j0 + 1, bbuf1, fbuf1)
        pltpu.async_copy(ones, mflat.at[fbuf1], sems1, add=True)
        scat(fbuf0, sems0).wait()
        scat(fbuf1, sems1).wait()
        plsc.subcore_barrier()
        pltpu.sync_copy(mflat.at[pl.ds(s * wpt, wpt)],
                        out_hbm.at[c, pl.ds(s * wpt, wpt)])

    return sc_kernel(packed2d, batch1d, zeros_flat)


def _onehot(b2d, bn, g):
    return (b2d == lax.broadcasted_iota(jnp.int32, (bn, g), 1)).astype(
        jnp.float32)


def _bcast_body(bn, g, h_ref, vn_ref, b_ref, o_ref):
    oh = _onehot(b_ref[...], bn, g)
    o_ref[...] = h_ref[...] + jnp.dot(oh, vn_ref[...],
                                      preferred_element_type=jnp.float32)


def _vn_broadcast(h, vn, b2d, BN):
    N, H = h.shape
    G = vn.shape[0]
    return pl.pallas_call(
        functools.partial(_bcast_body, BN, G),
        grid=(N // BN,),
        in_specs=[
            pl.BlockSpec((BN, H), lambda i: (i, 0)),
            pl.BlockSpec((G, H), lambda i: (0, 0)),
            pl.BlockSpec((BN, 1), lambda i: (i, 0)),
        ],
        out_specs=pl.BlockSpec((BN, H), lambda i: (i, 0)),
        out_shape=jax.ShapeDtypeStruct((N, H), jnp.float32),
    )(h, vn, b2d)


def _msum_body(a_ref, b_ref, o_ref):
    o_ref[...] = a_ref[...] + b_ref[...]


def _msum(a, b, BN):
    N, G = a.shape
    return pl.pallas_call(
        _msum_body,
        grid=(N // BN,),
        in_specs=[
            pl.BlockSpec((BN, G), lambda i: (i, 0)),
            pl.BlockSpec((BN, G), lambda i: (i, 0)),
        ],
        out_specs=pl.BlockSpec((BN, G), lambda i: (i, 0)),
        out_shape=jax.ShapeDtypeStruct((N, G), jnp.float32),
    )(a, b)


def _gin_common(bn, g, h_ref, p0_ref, p1_ref, b_ref, w1_ref,
                b1_ref, w2_ref, b2_ref, eps_ref):
    oh = _onehot(b_ref[...], bn, g)
    h_in = h_ref[...]
    aggr = p0_ref[...] + p1_ref[...]
    z = (1.0 + eps_ref[0, 0]) * h_in + aggr
    hid = jnp.maximum(
        jnp.dot(z, w1_ref[...], preferred_element_type=jnp.float32)
        + b1_ref[...], 0.0)
    z2 = jnp.dot(hid, w2_ref[...],
                 preferred_element_type=jnp.float32) + b2_ref[...]
    h_new = jnp.maximum(z2, 0.0) + h_in
    return h_in, h_new, oh


def _dense_body(bn, g, h_ref, p0_ref, p1_ref, b_ref, w1_ref,
                b1_ref, w2_ref, b2_ref, eps_ref, hn_ref, seg_ref):
    h_in, h_new, oh = _gin_common(bn, g, h_ref, p0_ref, p1_ref,
                                  b_ref, w1_ref, b1_ref, w2_ref, b2_ref,
                                  eps_ref)
    hn_ref[...] = h_new
    seg = jnp.dot(oh.T, h_in, preferred_element_type=jnp.float32)

    @pl.when(pl.program_id(0) == 0)
    def _():
        seg_ref[...] = seg

    @pl.when(pl.program_id(0) != 0)
    def _():
        seg_ref[...] += seg


def _gin_in_specs(BN, H, H2, G):
    return [
        pl.BlockSpec((BN, H), lambda i: (i, 0)),
        pl.BlockSpec((BN, H), lambda i: (i, 0)),
        pl.BlockSpec((BN, H), lambda i: (i, 0)),
        pl.BlockSpec((BN, 1), lambda i: (i, 0)),
        pl.BlockSpec((H, H2), lambda i: (0, 0)),
        pl.BlockSpec((H2,), lambda i: (0,)),
        pl.BlockSpec((H2, H), lambda i: (0, 0)),
        pl.BlockSpec((H,), lambda i: (0,)),
        pl.BlockSpec((1, 1), lambda i: (0, 0)),
    ]


def _gin_dense(h, p0, p1, b2d, w1, b1, w2, b2, epsl, G, BN):
    N, H = h.shape
    H2 = w1.shape[1]
    return pl.pallas_call(
        functools.partial(_dense_body, BN, G),
        grid=(N // BN,),
        in_specs=_gin_in_specs(BN, H, H2, G),
        out_specs=[
            pl.BlockSpec((BN, H), lambda i: (i, 0)),
            pl.BlockSpec((G, H), lambda i: (0, 0)),
        ],
        out_shape=[
            jax.ShapeDtypeStruct((N, H), jnp.float32),
            jax.ShapeDtypeStruct((G, H), jnp.float32),
        ],
    )(h, p0, p1, b2d, w1, b1, w2, b2, epsl)


def _last_body(bn, g, h_ref, p0_ref, p1_ref, b_ref, w1_ref,
               b1_ref, w2_ref, b2_ref, eps_ref, segmax_ref):
    _, h_new, _ = _gin_common(bn, g, h_ref, p0_ref, p1_ref,
                              b_ref, w1_ref, b1_ref, w2_ref, b2_ref, eps_ref)
    b2d = b_ref[...]
    first = pl.program_id(0) == 0

    def body(gi, carry):
        vals = jnp.where(b2d == gi, h_new, -jnp.inf)
        row = jnp.max(vals, axis=0, keepdims=True)
        cur = segmax_ref[pl.ds(gi, 1), :]
        segmax_ref[pl.ds(gi, 1), :] = jnp.where(first, row,
                                                jnp.maximum(cur, row))
        return carry

    lax.fori_loop(0, g, body, 0)


def _gin_last(h, p0, p1, b2d, w1, b1, w2, b2, epsl, G, BN):
    N, H = h.shape
    H2 = w1.shape[1]
    return pl.pallas_call(
        functools.partial(_last_body, BN, G),
        grid=(N // BN,),
        in_specs=_gin_in_specs(BN, H, H2, G),
        out_specs=pl.BlockSpec((G, H), lambda i: (0, 0)),
        out_shape=jax.ShapeDtypeStruct((G, H), jnp.float32),
    )(h, p0, p1, b2d, w1, b1, w2, b2, epsl)


def _vn_body(seg_ref, vn_ref, w1_ref, b1_ref, w2_ref, b2_ref, out_ref):
    vn = vn_ref[...]
    t = seg_ref[...] + vn
    t = jnp.maximum(
        jnp.dot(t, w1_ref[...], preferred_element_type=jnp.float32)
        + b1_ref[...], 0.0)
    out_ref[...] = vn + jnp.maximum(
        jnp.dot(t, w2_ref[...], preferred_element_type=jnp.float32)
        + b2_ref[...], 0.0)


def _vn_update(seg, vn, w1, b1, w2, b2):
    return pl.pallas_call(
        _vn_body,
        out_shape=jax.ShapeDtypeStruct(vn.shape, jnp.float32),
    )(seg, vn, w1, b1, w2, b2)


def _pred_body(hrep_ref, mg_ref, mc_ref, wa_ref, wb_ref, wc_ref, bp1_ref,
               wp2_ref, bp2_ref, out_ref):
    acc = (jnp.dot(hrep_ref[...], wa_ref[...],
                   preferred_element_type=jnp.float32)
           + jnp.dot(mg_ref[...], wb_ref[...],
                     preferred_element_type=jnp.float32)
           + jnp.dot(mc_ref[...], wc_ref[...],
                     preferred_element_type=jnp.float32)
           + bp1_ref[...])
    out_ref[...] = jnp.dot(jnp.maximum(acc, 0.0), wp2_ref[...],
                           preferred_element_type=jnp.float32) + bp2_ref[...]


def _predictor(hrep, morgan, maccs, wpa, wpb, wpc, bp1, wp2, bp2):
    G = hrep.shape[0]
    T = wp2.shape[1]
    return pl.pallas_call(
        _pred_body,
        out_shape=jax.ShapeDtypeStruct((G, T), jnp.float32),
    )(hrep, morgan, maccs, wpa, wpb, wpc, bp1, wp2, bp2)


def kernel(x, edge_index, batch, morgan, maccs, W1, b1, W2, b2, eps,
           vnW1, vnb1, vnW2, vnb2, Wp1, bp1, Wp2, bp2):
    N, H = x.shape
    E = edge_index.shape[1]
    G = morgan.shape[0]
    L = W1.shape[0]
    BN = 2000

    src = edge_index[0].astype(jnp.int32)
    dst = edge_index[1].astype(jnp.int32)
    b2d = batch.astype(jnp.int32).reshape(N, 1)

    per_tile = -(-E // NW)
    NB = -(-per_tile // K_EDGE)
    NB = -(-NB // 8) * 8
    E_pad = NW * NB * K_EDGE
    pad = E_pad - E
    pr = jnp.arange(pad, dtype=jnp.int32) % 8
    srcp = jnp.concatenate([src, pr])
    dstp = jnp.concatenate([dst, N + pr])
    zeros_nh = jnp.zeros((N, H), jnp.float32)

    vn = jnp.zeros((G, H), jnp.float32)
    h_in = x
    for l in range(L):
        if l > 0:
            h_in = _vn_broadcast(h_in, vn, b2d, BN)
        parts = _edge_aggregate(h_in, srcp, dstp, zeros_nh, N, H, NB)
        eps_l = eps[l].reshape(1, 1)
        if l < L - 1:
            h_new, seg = _gin_dense(h_in, parts[0], parts[1], b2d, W1[l],
                                    b1[l], W2[l], b2[l], eps_l, G, BN)
            vn = _vn_update(seg, vn, vnW1[l], vnb1[l], vnW2[l], vnb2[l])
            h_in = h_new
        else:
            hrep = _gin_last(h_in, parts[0], parts[1], b2d, W1[l], b1[l],
                             W2[l], b2[l], eps_l, G, BN)

    return _predictor(hrep, morgan, maccs, Wp1[:H], Wp1[H:H + 1024],
                      Wp1[H + 1024:], bp1, Wp2, bp2)


# speedup vs baseline: 1.2174x; 1.0706x over previous
import functools

import jax
import jax.numpy as jnp
from jax import lax
from jax.experimental import pallas as pl
from jax.experimental.pallas import tpu as pltpu
from jax.experimental.pallas import tpu_sc as plsc

NC = 2
NS = 16
NW = NC * NS
K_EDGE = 128


@functools.partial(jax.jit, static_argnums=(4, 5, 6))
def _edge_aggregate(h, srcp, dstp, zeros_nh, N, H, NB):
    npad = N + 8
    rpt = (N // NS) // 8 * 8
    tail0 = rpt * NS
    tail = N - tail0
    NP = NB // 2

    packed = (srcp | (dstp << 16)).reshape(NW * NB, K_EDGE)

    mesh = plsc.VectorSubcoreMesh(core_axis_name="c", subcore_axis_name="s")

    @functools.partial(
        pl.kernel,
        out_type=jax.ShapeDtypeStruct((NC, N, H), jnp.float32),
        mesh=mesh,
        scratch_types=[
            pltpu.VMEM((NB, K_EDGE), jnp.int32),
            pltpu.VMEM((K_EDGE,), jnp.int32),
            pltpu.VMEM((K_EDGE,), jnp.int32),
            pltpu.VMEM((K_EDGE,), jnp.int32),
            pltpu.VMEM((K_EDGE,), jnp.int32),
            pltpu.VMEM((K_EDGE, H), jnp.float32),
            pltpu.VMEM((K_EDGE, H), jnp.float32),
            pltpu.VMEM_SHARED((npad, H), jnp.float32),
            pltpu.SemaphoreType.DMA,
            pltpu.SemaphoreType.DMA,
            pltpu.SemaphoreType.DMA,
            pltpu.SemaphoreType.DMA,
        ],
    )
    def sc_kernel(h_hbm, pk_hbm, z_hbm, out_hbm, pidx, sidx0, didx0, sidx1,
                  didx1, rows0, rows1, aggr, sem0, sem1, sems0, sems1):
        c = lax.axis_index("c")
        s = lax.axis_index("s")
        wid = s * NC + c
        r0 = s * rpt
        zcp = pltpu.make_async_copy(z_hbm.at[pl.ds(r0, rpt)],
                                    aggr.at[pl.ds(r0, rpt)], sems0)
        zcp.start()
        if tail > 0:
            @pl.when(s == NS - 1)
            def _():
                pltpu.async_copy(z_hbm.at[pl.ds(tail0, tail)],
                                 aggr.at[pl.ds(tail0, tail)], sems1).wait()
        pltpu.sync_copy(pk_hbm.at[pl.ds(wid * NB, NB)], pidx)
        zcp.wait()
        plsc.subcore_barrier()

        def unpack(j, sb, db):
            for i in range(K_EDGE // 16):
                v = pidx[j, pl.ds(i * 16, 16)]
                sb[pl.ds(i * 16, 16)] = v & 0xFFFF
                db[pl.ds(i * 16, 16)] = jnp.right_shift(v, 16)

        def gather(sb, rows, sem):
            return pltpu.make_async_copy(h_hbm.at[sb], rows, sem)

        unpack(0, sidx0, didx0)
        unpack(1, sidx1, didx1)
        gather(sidx0, rows0, sem0).start()
        gather(sidx1, rows1, sem1).start()

        def body(p, carry):
            j0 = 2 * p
            gather(sidx0, rows0, sem0).wait()
            pltpu.sync_copy(rows0, aggr.at[didx0], add=True)
            unpack(j0 + 2, sidx0, didx0)
            gather(sidx0, rows0, sem0).start()
            gather(sidx1, rows1, sem1).wait()
            pltpu.sync_copy(rows1, aggr.at[didx1], add=True)
            unpack(j0 + 3, sidx1, didx1)
            gather(sidx1, rows1, sem1).start()
            return carry

        lax.fori_loop(0, NP - 1, body, 0)
        gather(sidx0, rows0, sem0).wait()
        pltpu.sync_copy(rows0, aggr.at[didx0], add=True)
        gather(sidx1, rows1, sem1).wait()
        pltpu.sync_copy(rows1, aggr.at[didx1], add=True)

        plsc.subcore_barrier()
        pltpu.sync_copy(aggr.at[pl.ds(r0, rpt)], out_hbm.at[c, pl.ds(r0, rpt)])
        if tail > 0:
            @pl.when(s == NS - 1)
            def _():
                pltpu.sync_copy(aggr.at[pl.ds(tail0, tail)],
                                out_hbm.at[c, pl.ds(tail0, tail)])

    return sc_kernel(h, packed, zeros_nh)


@functools.partial(jax.jit, static_argnums=(3, 4, 5))
def _count_matrix(packed2d, batch1d, zeros_flat, N, G, NB):
    npad = N + 8
    NG = N * G
    wpt = NG // NS

    mesh = plsc.VectorSubcoreMesh(core_axis_name="c", subcore_axis_name="s")

    NP = NB // 2
    scratch_types = [pltpu.VMEM((NB, K_EDGE), jnp.int32)]
    scratch_types += [pltpu.VMEM((K_EDGE,), jnp.int32)] * 6
    scratch_types += [
        pltpu.VMEM((K_EDGE,), jnp.float32),
        pltpu.VMEM_SHARED((npad * G,), jnp.float32),
    ]
    scratch_types += [pltpu.SemaphoreType.DMA] * 4

    @functools.partial(
        pl.kernel,
        out_type=jax.ShapeDtypeStruct((NC, NG), jnp.float32),
        mesh=mesh,
        scratch_types=scratch_types,
    )
    def sc_kernel(pk_hbm, b_hbm, z_hbm, out_hbm, pidx, sbuf0, sbuf1, bbuf0,
                  bbuf1, fbuf0, fbuf1, ones, mflat, semb0, semb1, sems0,
                  sems1):
        c = lax.axis_index("c")
        s = lax.axis_index("s")
        wid = s * NC + c
        pltpu.sync_copy(pk_hbm.at[pl.ds(wid * NB, NB)], pidx)
        pltpu.sync_copy(z_hbm.at[pl.ds(s * wpt, wpt)],
                        mflat.at[pl.ds(s * wpt, wpt)])
        for i in range(K_EDGE // 16):
            ones[pl.ds(i * 16, 16)] = jnp.full((16,), 1.0, jnp.float32)
        plsc.subcore_barrier()

        def unpack_s(j, sb):
            for i in range(K_EDGE // 16):
                v = pidx[j, pl.ds(i * 16, 16)]
                sb[pl.ds(i * 16, 16)] = v & 0xFFFF

        def fill_f(j, bb, fb):
            for i in range(K_EDGE // 16):
                v = pidx[j, pl.ds(i * 16, 16)]
                fb[pl.ds(i * 16, 16)] = (jnp.right_shift(v, 16) * G
                                         + bb[pl.ds(i * 16, 16)])

        def bgather(sb, bb, sem):
            return pltpu.make_async_copy(b_hbm.at[sb], bb, sem)

        def scat(fb, sem):
            return pltpu.make_async_copy(ones, mflat.at[fb], sem)

        unpack_s(0, sbuf0)
        bgather(sbuf0, bbuf0, semb0).start()
        unpack_s(1, sbuf1)
        bgather(sbuf1, bbuf1, semb1).start()

        def body(p, carry):
            j0 = 2 * p
            bgather(sbuf0, bbuf0, semb0).wait()
            fill_f(j0, bbuf0, fbuf0)
            pltpu.async_copy(ones, mflat.at[fbuf0], sems0, add=True)
            bgather(sbuf1, bbuf1, semb1).wait()
            fill_f(j0 + 1, bbuf1, fbuf1)
            pltpu.async_copy(ones, mflat.at[fbuf1], sems1, add=True)
            scat(fbuf0, sems0).wait()
            unpack_s(j0 + 2, sbuf0)
            bgather(sbuf0, bbuf0, semb0).start()
            scat(fbuf1, sems1).wait()
            unpack_s(j0 + 3, sbuf1)
            bgather(sbuf1, bbuf1, semb1).start()
            return carry

        lax.fori_loop(0, NP - 1, body, 0)
        j0 = NB - 2
        bgather(sbuf0, bbuf0, semb0).wait()
        fill_f(j0, bbuf0, fbuf0)
        pltpu.async_copy(ones, mflat.at[fbuf0], sems0, add=True)
        bgather(sbuf1, bbuf1, semb1).wait()
        fill_f(j0 + 1, bbuf1, fbuf1)
        pltpu.async_copy(ones, mflat.at[fbuf1], sems1, add=True)
        scat(fbuf0, sems0).wait()
        scat(fbuf1, sems1).wait()
        plsc.subcore_barrier()
        pltpu.sync_copy(mflat.at[pl.ds(s * wpt, wpt)],
                        out_hbm.at[c, pl.ds(s * wpt, wpt)])

    return sc_kernel(packed2d, batch1d, zeros_flat)


def _onehot(b2d, bn, g):
    return (b2d == lax.broadcasted_iota(jnp.int32, (bn, g), 1)).astype(
        jnp.float32)


def _bcast_body(bn, g, h_ref, vn_ref, b_ref, o_ref):
    oh = _onehot(b_ref[...], bn, g)
    o_ref[...] = h_ref[...] + jnp.dot(oh, vn_ref[...],
                                      preferred_element_type=jnp.float32)


def _vn_broadcast(h, vn, b2d, BN):
    N, H = h.shape
    G = vn.shape[0]
    return pl.pallas_call(
        functools.partial(_bcast_body, BN, G),
        grid=(N // BN,),
        in_specs=[
            pl.BlockSpec((BN, H), lambda i: (i, 0)),
            pl.BlockSpec((G, H), lambda i: (0, 0)),
            pl.BlockSpec((BN, 1), lambda i: (i, 0)),
        ],
        out_specs=pl.BlockSpec((BN, H), lambda i: (i, 0)),
        out_shape=jax.ShapeDtypeStruct((N, H), jnp.float32),
    )(h, vn, b2d)


def _msum_body(a_ref, b_ref, o_ref):
    o_ref[...] = a_ref[...] + b_ref[...]


def _msum(a, b, BN):
    N, G = a.shape
    return pl.pallas_call(
        _msum_body,
        grid=(N // BN,),
        in_specs=[
            pl.BlockSpec((BN, G), lambda i: (i, 0)),
            pl.BlockSpec((BN, G), lambda i: (i, 0)),
        ],
        out_specs=pl.BlockSpec((BN, G), lambda i: (i, 0)),
        out_shape=jax.ShapeDtypeStruct((N, G), jnp.float32),
    )(a, b)


def _gin_common(bn, g, h_ref, p0_ref, p1_ref, b_ref, w1_ref,
                b1_ref, w2_ref, b2_ref, eps_ref):
    oh = _onehot(b_ref[...], bn, g)
    h_in = h_ref[...]
    aggr = p0_ref[...] + p1_ref[...]
    z = (1.0 + eps_ref[0, 0]) * h_in + aggr
    hid = jnp.maximum(
        jnp.dot(z, w1_ref[...], preferred_element_type=jnp.float32)
        + b1_ref[...], 0.0)
    z2 = jnp.dot(hid, w2_ref[...],
                 preferred_element_type=jnp.float32) + b2_ref[...]
    h_new = jnp.maximum(z2, 0.0) + h_in
    return h_in, h_new, oh


def _dense_body(bn, g, h_ref, p0_ref, p1_ref, b_ref, w1_ref,
                b1_ref, w2_ref, b2_ref, eps_ref, vn_ref, vw1_ref, vb1_ref,
                vw2_ref, vb2_ref, hn_ref, vno_ref, seg_ref):
    h_in, h_new, oh = _gin_common(bn, g, h_ref, p0_ref, p1_ref,
                                  b_ref, w1_ref, b1_ref, w2_ref, b2_ref,
                                  eps_ref)
    hn_ref[...] = h_new
    seg = jnp.dot(oh.T, h_in, preferred_element_type=jnp.float32)

    @pl.when(pl.program_id(0) == 0)
    def _():
        seg_ref[...] = seg

    @pl.when(pl.program_id(0) != 0)
    def _():
        seg_ref[...] += seg

    @pl.when(pl.program_id(0) == pl.num_programs(0) - 1)
    def _():
        vn = vn_ref[...]
        t = seg_ref[...] + vn
        t = jnp.maximum(
            jnp.dot(t, vw1_ref[...], preferred_element_type=jnp.float32)
            + vb1_ref[...], 0.0)
        vno_ref[...] = vn + jnp.maximum(
            jnp.dot(t, vw2_ref[...], preferred_element_type=jnp.float32)
            + vb2_ref[...], 0.0)


def _gin_in_specs(BN, H, H2, G):
    return [
        pl.BlockSpec((BN, H), lambda i: (i, 0)),
        pl.BlockSpec((BN, H), lambda i: (i, 0)),
        pl.BlockSpec((BN, H), lambda i: (i, 0)),
        pl.BlockSpec((BN, 1), lambda i: (i, 0)),
        pl.BlockSpec((H, H2), lambda i: (0, 0)),
        pl.BlockSpec((H2,), lambda i: (0,)),
        pl.BlockSpec((H2, H), lambda i: (0, 0)),
        pl.BlockSpec((H,), lambda i: (0,)),
        pl.BlockSpec((1, 1), lambda i: (0, 0)),
    ]


def _gin_dense(h, p0, p1, b2d, w1, b1, w2, b2, epsl, vn, vw1, vb1, vw2, vb2,
               G, BN):
    N, H = h.shape
    H2 = w1.shape[1]
    return pl.pallas_call(
        functools.partial(_dense_body, BN, G),
        grid=(N // BN,),
        in_specs=_gin_in_specs(BN, H, H2, G) + [
            pl.BlockSpec((G, H), lambda i: (0, 0)),
            pl.BlockSpec((H, H2), lambda i: (0, 0)),
            pl.BlockSpec((H2,), lambda i: (0,)),
            pl.BlockSpec((H2, H), lambda i: (0, 0)),
            pl.BlockSpec((H,), lambda i: (0,)),
        ],
        out_specs=[
            pl.BlockSpec((BN, H), lambda i: (i, 0)),
            pl.BlockSpec((G, H), lambda i: (0, 0)),
        ],
        out_shape=[
            jax.ShapeDtypeStruct((N, H), jnp.float32),
            jax.ShapeDtypeStruct((G, H), jnp.float32),
        ],
        scratch_shapes=[pltpu.VMEM((G, H), jnp.float32)],
    )(h, p0, p1, b2d, w1, b1, w2, b2, epsl, vn, vw1, vb1, vw2, vb2)


def _last_body(bn, g, h_ref, p0_ref, p1_ref, b_ref, w1_ref,
               b1_ref, w2_ref, b2_ref, eps_ref, mg_ref, mc_ref, wpa_ref,
               wpb_ref, wpc_ref, bp1_ref, wp2_ref, bp2_ref, out_ref,
               segmax_ref):
    _, h_new, _ = _gin_common(bn, g, h_ref, p0_ref, p1_ref,
                              b_ref, w1_ref, b1_ref, w2_ref, b2_ref, eps_ref)
    b2d = b_ref[...]
    first = pl.program_id(0) == 0

    def body(gi, carry):
        vals = jnp.where(b2d == gi, h_new, -jnp.inf)
        row = jnp.max(vals, axis=0, keepdims=True)
        cur = segmax_ref[pl.ds(gi, 1), :]
        segmax_ref[pl.ds(gi, 1), :] = jnp.where(first, row,
                                                jnp.maximum(cur, row))
        return carry

    lax.fori_loop(0, g, body, 0)

    @pl.when(pl.program_id(0) == pl.num_programs(0) - 1)
    def _():
        acc = (jnp.dot(segmax_ref[...], wpa_ref[...],
                       preferred_element_type=jnp.float32)
               + jnp.dot(mg_ref[...], wpb_ref[...],
                         preferred_element_type=jnp.float32)
               + jnp.dot(mc_ref[...], wpc_ref[...],
                         preferred_element_type=jnp.float32)
               + bp1_ref[...])
        out_ref[...] = jnp.dot(
            jnp.maximum(acc, 0.0), wp2_ref[...],
            preferred_element_type=jnp.float32) + bp2_ref[...]


def _gin_last(h, p0, p1, b2d, w1, b1, w2, b2, epsl, morgan, maccs, wpa, wpb,
              wpc, bp1, wp2, bp2, G, BN):
    N, H = h.shape
    H2 = w1.shape[1]
    DM = morgan.shape[1]
    DC = maccs.shape[1]
    T = wp2.shape[1]
    P2 = wp2.shape[0]
    return pl.pallas_call(
        functools.partial(_last_body, BN, G),
        grid=(N // BN,),
        in_specs=_gin_in_specs(BN, H, H2, G) + [
            pl.BlockSpec((G, DM), lambda i: (0, 0)),
            pl.BlockSpec((G, DC), lambda i: (0, 0)),
            pl.BlockSpec((H, P2), lambda i: (0, 0)),
            pl.BlockSpec((DM, P2), lambda i: (0, 0)),
            pl.BlockSpec((DC, P2), lambda i: (0, 0)),
            pl.BlockSpec((P2,), lambda i: (0,)),
            pl.BlockSpec((P2, T), lambda i: (0, 0)),
            pl.BlockSpec((T,), lambda i: (0,)),
        ],
        out_specs=pl.BlockSpec((G, T), lambda i: (0, 0)),
        out_shape=jax.ShapeDtypeStruct((G, T), jnp.float32),
        scratch_shapes=[pltpu.VMEM((G, H), jnp.float32)],
    )(h, p0, p1, b2d, w1, b1, w2, b2, epsl, morgan, maccs, wpa, wpb, wpc,
      bp1, wp2, bp2)


def _vn_body(seg_ref, vn_ref, w1_ref, b1_ref, w2_ref, b2_ref, out_ref):
    vn = vn_ref[...]
    t = seg_ref[...] + vn
    t = jnp.maximum(
        jnp.dot(t, w1_ref[...], preferred_element_type=jnp.float32)
        + b1_ref[...], 0.0)
    out_ref[...] = vn + jnp.maximum(
        jnp.dot(t, w2_ref[...], preferred_element_type=jnp.float32)
        + b2_ref[...], 0.0)


def _vn_update(seg, vn, w1, b1, w2, b2):
    return pl.pallas_call(
        _vn_body,
        out_shape=jax.ShapeDtypeStruct(vn.shape, jnp.float32),
    )(seg, vn, w1, b1, w2, b2)


def _pred_body(hrep_ref, mg_ref, mc_ref, wa_ref, wb_ref, wc_ref, bp1_ref,
               wp2_ref, bp2_ref, out_ref):
    acc = (jnp.dot(hrep_ref[...], wa_ref[...],
                   preferred_element_type=jnp.float32)
           + jnp.dot(mg_ref[...], wb_ref[...],
                     preferred_element_type=jnp.float32)
           + jnp.dot(mc_ref[...], wc_ref[...],
                     preferred_element_type=jnp.float32)
           + bp1_ref[...])
    out_ref[...] = jnp.dot(jnp.maximum(acc, 0.0), wp2_ref[...],
                           preferred_element_type=jnp.float32) + bp2_ref[...]


def _predictor(hrep, morgan, maccs, wpa, wpb, wpc, bp1, wp2, bp2):
    G = hrep.shape[0]
    T = wp2.shape[1]
    return pl.pallas_call(
        _pred_body,
        out_shape=jax.ShapeDtypeStruct((G, T), jnp.float32),
    )(hrep, morgan, maccs, wpa, wpb, wpc, bp1, wp2, bp2)


def kernel(x, edge_index, batch, morgan, maccs, W1, b1, W2, b2, eps,
           vnW1, vnb1, vnW2, vnb2, Wp1, bp1, Wp2, bp2):
    N, H = x.shape
    E = edge_index.shape[1]
    G = morgan.shape[0]
    L = W1.shape[0]
    BN = 5000

    src = edge_index[0].astype(jnp.int32)
    dst = edge_index[1].astype(jnp.int32)
    b2d = batch.astype(jnp.int32).reshape(N, 1)

    per_tile = -(-E // NW)
    NB = -(-per_tile // K_EDGE)
    NB = -(-NB // 8) * 8
    E_pad = NW * NB * K_EDGE
    pad = E_pad - E
    pr = jnp.arange(pad, dtype=jnp.int32) % 8
    srcp = jnp.concatenate([src, pr])
    dstp = jnp.concatenate([dst, N + pr])
    zeros_nh = jnp.zeros((N, H), jnp.float32)

    vn = jnp.zeros((G, H), jnp.float32)
    DM = morgan.shape[1]
    h_in = x
    for l in range(L):
        if l > 0:
            h_in = _vn_broadcast(h_in, vn, b2d, BN)
        parts = _edge_aggregate(h_in, srcp, dstp, zeros_nh, N, H, NB)
        eps_l = eps[l].reshape(1, 1)
        if l < L - 1:
            h_in, vn = _gin_dense(h_in, parts[0], parts[1], b2d, W1[l],
                                  b1[l], W2[l], b2[l], eps_l, vn, vnW1[l],
                                  vnb1[l], vnW2[l], vnb2[l], G, BN)
        else:
            return _gin_last(h_in, parts[0], parts[1], b2d, W1[l], b1[l],
                             W2[l], b2[l], eps_l, morgan, maccs, Wp1[:H],
                             Wp1[H:H + DM], Wp1[H + DM:], bp1, Wp2, bp2,
                             G, BN)


# speedup vs baseline: 1.2222x; 1.0039x over previous
import functools

import jax
import jax.numpy as jnp
from jax import lax
from jax.experimental import pallas as pl
from jax.experimental.pallas import tpu as pltpu
from jax.experimental.pallas import tpu_sc as plsc

NC = 2
NS = 16
NW = NC * NS
K_EDGE = 128


@functools.partial(jax.jit, static_argnums=(4, 5, 6))
def _edge_aggregate(h, srcp, dstp, zeros_nh, N, H, NB):
    npad = N + 8
    rpt = (N // NS) // 8 * 8
    tail0 = rpt * NS
    tail = N - tail0
    NP = NB // 2

    packed = (srcp | (dstp << 16)).reshape(NW * NB, K_EDGE)

    mesh = plsc.VectorSubcoreMesh(core_axis_name="c", subcore_axis_name="s")

    @functools.partial(
        pl.kernel,
        out_type=jax.ShapeDtypeStruct((NC, N, H), jnp.float32),
        mesh=mesh,
        scratch_types=[
            pltpu.VMEM((NB, K_EDGE), jnp.int32),
            pltpu.VMEM((K_EDGE,), jnp.int32),
            pltpu.VMEM((K_EDGE,), jnp.int32),
            pltpu.VMEM((K_EDGE,), jnp.int32),
            pltpu.VMEM((K_EDGE,), jnp.int32),
            pltpu.VMEM((K_EDGE, H), jnp.float32),
            pltpu.VMEM((K_EDGE, H), jnp.float32),
            pltpu.VMEM_SHARED((npad, H), jnp.float32),
            pltpu.SemaphoreType.DMA,
            pltpu.SemaphoreType.DMA,
            pltpu.SemaphoreType.DMA,
            pltpu.SemaphoreType.DMA,
        ],
    )
    def sc_kernel(h_hbm, pk_hbm, z_hbm, out_hbm, pidx, sidx0, didx0, sidx1,
                  didx1, rows0, rows1, aggr, sem0, sem1, sems0, sems1):
        c = lax.axis_index("c")
        s = lax.axis_index("s")
        wid = s * NC + c
        r0 = s * rpt
        zcp = pltpu.make_async_copy(z_hbm.at[pl.ds(r0, rpt)],
                                    aggr.at[pl.ds(r0, rpt)], sems0)
        zcp.start()
        if tail > 0:
            @pl.when(s == NS - 1)
            def _():
                pltpu.async_copy(z_hbm.at[pl.ds(tail0, tail)],
                                 aggr.at[pl.ds(tail0, tail)], sems1).wait()
        pltpu.sync_copy(pk_hbm.at[pl.ds(wid * NB, NB)], pidx)

        def unpack(j, sb, db):
            for i in range(K_EDGE // 16):
                v = pidx[j, pl.ds(i * 16, 16)]
                sb[pl.ds(i * 16, 16)] = v & 0xFFFF
                db[pl.ds(i * 16, 16)] = jnp.right_shift(v, 16)

        def gather(sb, rows, sem):
            return pltpu.make_async_copy(h_hbm.at[sb], rows, sem)

        unpack(0, sidx0, didx0)
        unpack(1, sidx1, didx1)
        gather(sidx0, rows0, sem0).start()
        gather(sidx1, rows1, sem1).start()
        zcp.wait()
        plsc.subcore_barrier()

        def body(p, carry):
            j0 = 2 * p
            gather(sidx0, rows0, sem0).wait()
            pltpu.sync_copy(rows0, aggr.at[didx0], add=True)
            unpack(j0 + 2, sidx0, didx0)
            gather(sidx0, rows0, sem0).start()
            gather(sidx1, rows1, sem1).wait()
            pltpu.sync_copy(rows1, aggr.at[didx1], add=True)
            unpack(j0 + 3, sidx1, didx1)
            gather(sidx1, rows1, sem1).start()
            return carry

        lax.fori_loop(0, NP - 1, body, 0)
        gather(sidx0, rows0, sem0).wait()
        pltpu.sync_copy(rows0, aggr.at[didx0], add=True)
        gather(sidx1, rows1, sem1).wait()
        pltpu.sync_copy(rows1, aggr.at[didx1], add=True)

        plsc.subcore_barrier()
        pltpu.sync_copy(aggr.at[pl.ds(r0, rpt)], out_hbm.at[c, pl.ds(r0, rpt)])
        if tail > 0:
            @pl.when(s == NS - 1)
            def _():
                pltpu.sync_copy(aggr.at[pl.ds(tail0, tail)],
                                out_hbm.at[c, pl.ds(tail0, tail)])

    return sc_kernel(h, packed, zeros_nh)


@functools.partial(jax.jit, static_argnums=(3, 4, 5))
def _count_matrix(packed2d, batch1d, zeros_flat, N, G, NB):
    npad = N + 8
    NG = N * G
    wpt = NG // NS

    mesh = plsc.VectorSubcoreMesh(core_axis_name="c", subcore_axis_name="s")

    NP = NB // 2
    scratch_types = [pltpu.VMEM((NB, K_EDGE), jnp.int32)]
    scratch_types += [pltpu.VMEM((K_EDGE,), jnp.int32)] * 6
    scratch_types += [
        pltpu.VMEM((K_EDGE,), jnp.float32),
        pltpu.VMEM_SHARED((npad * G,), jnp.float32),
    ]
    scratch_types += [pltpu.SemaphoreType.DMA] * 4

    @functools.partial(
        pl.kernel,
        out_type=jax.ShapeDtypeStruct((NC, NG), jnp.float32),
        mesh=mesh,
        scratch_types=scratch_types,
    )
    def sc_kernel(pk_hbm, b_hbm, z_hbm, out_hbm, pidx, sbuf0, sbuf1, bbuf0,
                  bbuf1, fbuf0, fbuf1, ones, mflat, semb0, semb1, sems0,
                  sems1):
        c = lax.axis_index("c")
        s = lax.axis_index("s")
        wid = s * NC + c
        pltpu.sync_copy(pk_hbm.at[pl.ds(wid * NB, NB)], pidx)
        pltpu.sync_copy(z_hbm.at[pl.ds(s * wpt, wpt)],
                        mflat.at[pl.ds(s * wpt, wpt)])
        for i in range(K_EDGE // 16):
            ones[pl.ds(i * 16, 16)] = jnp.full((16,), 1.0, jnp.float32)
        plsc.subcore_barrier()

        def unpack_s(j, sb):
            for i in range(K_EDGE // 16):
                v = pidx[j, pl.ds(i * 16, 16)]
                sb[pl.ds(i * 16, 16)] = v & 0xFFFF

        def fill_f(j, bb, fb):
            for i in range(K_EDGE // 16):
                v = pidx[j, pl.ds(i * 16, 16)]
                fb[pl.ds(i * 16, 16)] = (jnp.right_shift(v, 16) * G
                                         + bb[pl.ds(i * 16, 16)])

        def bgather(sb, bb, sem):
            return pltpu.make_async_copy(b_hbm.at[sb], bb, sem)

        def scat(fb, sem):
            return pltpu.make_async_copy(ones, mflat.at[fb], sem)

        unpack_s(0, sbuf0)
        bgather(sbuf0, bbuf0, semb0).start()
        unpack_s(1, sbuf1)
        bgather(sbuf1, bbuf1, semb1).start()

        def body(p, carry):
            j0 = 2 * p
            bgather(sbuf0, bbuf0, semb0).wait()
            fill_f(j0, bbuf0, fbuf0)
            pltpu.async_copy(ones, mflat.at[fbuf0], sems0, add=True)
            bgather(sbuf1, bbuf1, semb1).wait()
            fill_f(j0 + 1, bbuf1, fbuf1)
            pltpu.async_copy(ones, mflat.at[fbuf1], sems1, add=True)
            scat(fbuf0, sems0).wait()
            unpack_s(j0 + 2, sbuf0)
            bgather(sbuf0, bbuf0, semb0).start()
            scat(fbuf1, sems1).wait()
            unpack_s(j0 + 3, sbuf1)
            bgather(sbuf1, bbuf1, semb1).start()
            return carry

        lax.fori_loop(0, NP - 1, body, 0)
        j0 = NB - 2
        bgather(sbuf0, bbuf0, semb0).wait()
        fill_f(j0, bbuf0, fbuf0)
        pltpu.async_copy(ones, mflat.at[fbuf0], sems0, add=True)
        bgather(sbuf1, bbuf1, semb1).wait()
        fill_f(j0 + 1, bbuf1, fbuf1)
        pltpu.async_copy(ones, mflat.at[fbuf1], sems1, add=True)
        scat(fbuf0, sems0).wait()
        scat(fbuf1, sems1).wait()
        plsc.subcore_barrier()
        pltpu.sync_copy(mflat.at[pl.ds(s * wpt, wpt)],
                        out_hbm.at[c, pl.ds(s * wpt, wpt)])

    return sc_kernel(packed2d, batch1d, zeros_flat)


def _onehot(b2d, bn, g):
    return (b2d == lax.broadcasted_iota(jnp.int32, (bn, g), 1)).astype(
        jnp.float32)


def _bcast_body(bn, g, h_ref, vn_ref, b_ref, o_ref):
    oh = _onehot(b_ref[...], bn, g)
    o_ref[...] = h_ref[...] + jnp.dot(oh, vn_ref[...],
                                      preferred_element_type=jnp.float32)


def _vn_broadcast(h, vn, b2d, BN):
    N, H = h.shape
    G = vn.shape[0]
    return pl.pallas_call(
        functools.partial(_bcast_body, BN, G),
        grid=(N // BN,),
        in_specs=[
            pl.BlockSpec((BN, H), lambda i: (i, 0)),
            pl.BlockSpec((G, H), lambda i: (0, 0)),
            pl.BlockSpec((BN, 1), lambda i: (i, 0)),
        ],
        out_specs=pl.BlockSpec((BN, H), lambda i: (i, 0)),
        out_shape=jax.ShapeDtypeStruct((N, H), jnp.float32),
    )(h, vn, b2d)


def _msum_body(a_ref, b_ref, o_ref):
    o_ref[...] = a_ref[...] + b_ref[...]


def _msum(a, b, BN):
    N, G = a.shape
    return pl.pallas_call(
        _msum_body,
        grid=(N // BN,),
        in_specs=[
            pl.BlockSpec((BN, G), lambda i: (i, 0)),
            pl.BlockSpec((BN, G), lambda i: (i, 0)),
        ],
        out_specs=pl.BlockSpec((BN, G), lambda i: (i, 0)),
        out_shape=jax.ShapeDtypeStruct((N, G), jnp.float32),
    )(a, b)


def _gin_common(bn, g, h_ref, p0_ref, p1_ref, b_ref, w1_ref,
                b1_ref, w2_ref, b2_ref, eps_ref):
    oh = _onehot(b_ref[...], bn, g)
    h_in = h_ref[...]
    aggr = p0_ref[...] + p1_ref[...]
    z = (1.0 + eps_ref[0, 0]) * h_in + aggr
    hid = jnp.maximum(
        jnp.dot(z, w1_ref[...], preferred_element_type=jnp.float32)
        + b1_ref[...], 0.0)
    z2 = jnp.dot(hid, w2_ref[...],
                 preferred_element_type=jnp.float32) + b2_ref[...]
    h_new = jnp.maximum(z2, 0.0) + h_in
    return h_in, h_new, oh


def _dense_body(bn, g, h_ref, p0_ref, p1_ref, b_ref, w1_ref,
                b1_ref, w2_ref, b2_ref, eps_ref, vn_ref, vw1_ref, vb1_ref,
                vw2_ref, vb2_ref, hn_ref, vno_ref, seg_ref):
    h_in, h_new, oh = _gin_common(bn, g, h_ref, p0_ref, p1_ref,
                                  b_ref, w1_ref, b1_ref, w2_ref, b2_ref,
                                  eps_ref)
    hn_ref[...] = h_new
    seg = jnp.dot(oh.T, h_in, preferred_element_type=jnp.float32)

    @pl.when(pl.program_id(0) == 0)
    def _():
        seg_ref[...] = seg

    @pl.when(pl.program_id(0) != 0)
    def _():
        seg_ref[...] += seg

    @pl.when(pl.program_id(0) == pl.num_programs(0) - 1)
    def _():
        vn = vn_ref[...]
        t = seg_ref[...] + vn
        t = jnp.maximum(
            jnp.dot(t, vw1_ref[...], preferred_element_type=jnp.float32)
            + vb1_ref[...], 0.0)
        vno_ref[...] = vn + jnp.maximum(
            jnp.dot(t, vw2_ref[...], preferred_element_type=jnp.float32)
            + vb2_ref[...], 0.0)


def _gin_in_specs(BN, H, H2, G):
    return [
        pl.BlockSpec((BN, H), lambda i: (i, 0)),
        pl.BlockSpec((BN, H), lambda i: (i, 0)),
        pl.BlockSpec((BN, H), lambda i: (i, 0)),
        pl.BlockSpec((BN, 1), lambda i: (i, 0)),
        pl.BlockSpec((H, H2), lambda i: (0, 0)),
        pl.BlockSpec((H2,), lambda i: (0,)),
        pl.BlockSpec((H2, H), lambda i: (0, 0)),
        pl.BlockSpec((H,), lambda i: (0,)),
        pl.BlockSpec((1, 1), lambda i: (0, 0)),
    ]


def _gin_dense(h, p0, p1, b2d, w1, b1, w2, b2, epsl, vn, vw1, vb1, vw2, vb2,
               G, BN):
    N, H = h.shape
    H2 = w1.shape[1]
    return pl.pallas_call(
        functools.partial(_dense_body, BN, G),
        grid=(N // BN,),
        in_specs=_gin_in_specs(BN, H, H2, G) + [
            pl.BlockSpec((G, H), lambda i: (0, 0)),
            pl.BlockSpec((H, H2), lambda i: (0, 0)),
            pl.BlockSpec((H2,), lambda i: (0,)),
            pl.BlockSpec((H2, H), lambda i: (0, 0)),
            pl.BlockSpec((H,), lambda i: (0,)),
        ],
        out_specs=[
            pl.BlockSpec((BN, H), lambda i: (i, 0)),
            pl.BlockSpec((G, H), lambda i: (0, 0)),
        ],
        out_shape=[
            jax.ShapeDtypeStruct((N, H), jnp.float32),
            jax.ShapeDtypeStruct((G, H), jnp.float32),
        ],
        scratch_shapes=[pltpu.VMEM((G, H), jnp.float32)],
    )(h, p0, p1, b2d, w1, b1, w2, b2, epsl, vn, vw1, vb1, vw2, vb2)


def _last_body(bn, g, h_ref, p0_ref, p1_ref, b_ref, w1_ref,
               b1_ref, w2_ref, b2_ref, eps_ref, mg_ref, mc_ref, wpa_ref,
               wpb_ref, wpc_ref, bp1_ref, wp2_ref, bp2_ref, out_ref,
               segmax_ref):
    _, h_new, _ = _gin_common(bn, g, h_ref, p0_ref, p1_ref,
                              b_ref, w1_ref, b1_ref, w2_ref, b2_ref, eps_ref)
    b2d = b_ref[...]
    first = pl.program_id(0) == 0

    def body(gi, carry):
        vals = jnp.where(b2d == gi, h_new, -jnp.inf)
        row = jnp.max(vals, axis=0, keepdims=True)
        cur = segmax_ref[pl.ds(gi, 1), :]
        segmax_ref[pl.ds(gi, 1), :] = jnp.where(first, row,
                                                jnp.maximum(cur, row))
        return carry

    lax.fori_loop(0, g, body, 0)

    @pl.when(pl.program_id(0) == pl.num_programs(0) - 1)
    def _():
        acc = (jnp.dot(segmax_ref[...], wpa_ref[...],
                       preferred_element_type=jnp.float32)
               + jnp.dot(mg_ref[...], wpb_ref[...],
                         preferred_element_type=jnp.float32)
               + jnp.dot(mc_ref[...], wpc_ref[...],
                         preferred_element_type=jnp.float32)
               + bp1_ref[...])
        out_ref[...] = jnp.dot(
            jnp.maximum(acc, 0.0), wp2_ref[...],
            preferred_element_type=jnp.float32) + bp2_ref[...]


def _gin_last(h, p0, p1, b2d, w1, b1, w2, b2, epsl, morgan, maccs, wpa, wpb,
              wpc, bp1, wp2, bp2, G, BN):
    N, H = h.shape
    H2 = w1.shape[1]
    DM = morgan.shape[1]
    DC = maccs.shape[1]
    T = wp2.shape[1]
    P2 = wp2.shape[0]
    return pl.pallas_call(
        functools.partial(_last_body, BN, G),
        grid=(N // BN,),
        in_specs=_gin_in_specs(BN, H, H2, G) + [
            pl.BlockSpec((G, DM), lambda i: (0, 0)),
            pl.BlockSpec((G, DC), lambda i: (0, 0)),
            pl.BlockSpec((H, P2), lambda i: (0, 0)),
            pl.BlockSpec((DM, P2), lambda i: (0, 0)),
            pl.BlockSpec((DC, P2), lambda i: (0, 0)),
            pl.BlockSpec((P2,), lambda i: (0,)),
            pl.BlockSpec((P2, T), lambda i: (0, 0)),
            pl.BlockSpec((T,), lambda i: (0,)),
        ],
        out_specs=pl.BlockSpec((G, T), lambda i: (0, 0)),
        out_shape=jax.ShapeDtypeStruct((G, T), jnp.float32),
        scratch_shapes=[pltpu.VMEM((G, H), jnp.float32)],
    )(h, p0, p1, b2d, w1, b1, w2, b2, epsl, morgan, maccs, wpa, wpb, wpc,
      bp1, wp2, bp2)


def _vn_body(seg_ref, vn_ref, w1_ref, b1_ref, w2_ref, b2_ref, out_ref):
    vn = vn_ref[...]
    t = seg_ref[...] + vn
    t = jnp.maximum(
        jnp.dot(t, w1_ref[...], preferred_element_type=jnp.float32)
        + b1_ref[...], 0.0)
    out_ref[...] = vn + jnp.maximum(
        jnp.dot(t, w2_ref[...], preferred_element_type=jnp.float32)
        + b2_ref[...], 0.0)


def _vn_update(seg, vn, w1, b1, w2, b2):
    return pl.pallas_call(
        _vn_body,
        out_shape=jax.ShapeDtypeStruct(vn.shape, jnp.float32),
    )(seg, vn, w1, b1, w2, b2)


def _pred_body(hrep_ref, mg_ref, mc_ref, wa_ref, wb_ref, wc_ref, bp1_ref,
               wp2_ref, bp2_ref, out_ref):
    acc = (jnp.dot(hrep_ref[...], wa_ref[...],
                   preferred_element_type=jnp.float32)
           + jnp.dot(mg_ref[...], wb_ref[...],
                     preferred_element_type=jnp.float32)
           + jnp.dot(mc_ref[...], wc_ref[...],
                     preferred_element_type=jnp.float32)
           + bp1_ref[...])
    out_ref[...] = jnp.dot(jnp.maximum(acc, 0.0), wp2_ref[...],
                           preferred_element_type=jnp.float32) + bp2_ref[...]


def _predictor(hrep, morgan, maccs, wpa, wpb, wpc, bp1, wp2, bp2):
    G = hrep.shape[0]
    T = wp2.shape[1]
    return pl.pallas_call(
        _pred_body,
        out_shape=jax.ShapeDtypeStruct((G, T), jnp.float32),
    )(hrep, morgan, maccs, wpa, wpb, wpc, bp1, wp2, bp2)


def kernel(x, edge_index, batch, morgan, maccs, W1, b1, W2, b2, eps,
           vnW1, vnb1, vnW2, vnb2, Wp1, bp1, Wp2, bp2):
    N, H = x.shape
    E = edge_index.shape[1]
    G = morgan.shape[0]
    L = W1.shape[0]
    BN = 5000

    src = edge_index[0].astype(jnp.int32)
    dst = edge_index[1].astype(jnp.int32)
    b2d = batch.astype(jnp.int32).reshape(N, 1)

    per_tile = -(-E // NW)
    NB = -(-per_tile // K_EDGE)
    NB = -(-NB // 8) * 8
    E_pad = NW * NB * K_EDGE
    pad = E_pad - E
    pr = jnp.arange(pad, dtype=jnp.int32) % 8
    srcp = jnp.concatenate([src, pr])
    dstp = jnp.concatenate([dst, N + pr])
    zeros_nh = jnp.zeros((N, H), jnp.float32)

    vn = jnp.zeros((G, H), jnp.float32)
    DM = morgan.shape[1]
    h_in = x
    for l in range(L):
        if l > 0:
            h_in = _vn_broadcast(h_in, vn, b2d, BN)
        parts = _edge_aggregate(h_in, srcp, dstp, zeros_nh, N, H, NB)
        eps_l = eps[l].reshape(1, 1)
        if l < L - 1:
            h_in, vn = _gin_dense(h_in, parts[0], parts[1], b2d, W1[l],
                                  b1[l], W2[l], b2[l], eps_l, vn, vnW1[l],
                                  vnb1[l], vnW2[l], vnb2[l], G, BN)
        else:
            return _gin_last(h_in, parts[0], parts[1], b2d, W1[l], b1[l],
                             W2[l], b2[l], eps_l, morgan, maccs, Wp1[:H],
                             Wp1[H:H + DM], Wp1[H + DM:], bp1, Wp2, bp2,
                             G, BN)
